# Initial kernel scaffold; baseline (speedup 1.0000x reference)
import jax
import jax.numpy as jnp
from jax.experimental import pallas as pl


def kernel(inputs, idf, weights):
    raise NotImplementedError("write your pallas kernel here")


# speedup vs baseline: 2.9058x; 2.9058x over previous
import functools

import jax
import jax.numpy as jnp
from jax import lax
from jax.experimental import pallas as pl
from jax.experimental.pallas import tpu as pltpu
from jax.experimental.pallas import tpu_sc as plsc

B, L, V, D = 16384, 50, 1000000, 32

NC, NS = 2, 16
NW = NC * NS
BPW = B // NW
ROWS_PER_GATHER = 100
CHUNK_B = 16
CHUNK_ROWS = CHUNK_B * L
GATHERS_PER_CHUNK = CHUNK_ROWS // ROWS_PER_GATHER
NCHUNKS = BPW // CHUNK_B
GPW = BPW * L // ROWS_PER_GATHER

_mesh = plsc.VectorSubcoreMesh(core_axis_name="c", subcore_axis_name="s")


@functools.partial(
    pl.kernel,
    out_type=jax.ShapeDtypeStruct((B, D), jnp.float32),
    mesh=_mesh,
    compiler_params=pltpu.CompilerParams(needs_layout_passes=False,
                                         use_tc_tiling_on_sc=False),
    scratch_types=[
        pltpu.VMEM((GPW, ROWS_PER_GATHER), jnp.int32),
        pltpu.VMEM((BPW * L,), jnp.float32),
        pltpu.VMEM((CHUNK_ROWS, D), jnp.float32),
        pltpu.VMEM((CHUNK_ROWS, D), jnp.float32),
        pltpu.VMEM((BPW, D), jnp.float32),
        pltpu.VMEM((16,), jnp.float32),
        pltpu.SemaphoreType.DMA,
        pltpu.SemaphoreType.DMA,
    ],
)
def _sc_embed(idx_hbm, idf_hbm, w_hbm, out_hbm,
              idx_v, idf_v, rows_a, rows_b, out_v, s_buf, sem_a, sem_b):
    wid = lax.axis_index("s") * NC + lax.axis_index("c")

    pltpu.sync_copy(idx_hbm.at[pl.ds(wid * GPW, GPW)], idx_v)
    pltpu.sync_copy(idf_hbm.at[pl.ds(wid * (BPW * L), BPW * L)], idf_v)

    def issue(g, buf, sem):
        for jj in range(GATHERS_PER_CHUNK):
            pltpu.async_copy(
                w_hbm.at[idx_v.at[g * GATHERS_PER_CHUNK + jj]],
                buf.at[pl.ds(jj * ROWS_PER_GATHER, ROWS_PER_GATHER)],
                sem,
            )

    def drain(buf, sem):
        pltpu.make_async_copy(w_hbm.at[pl.ds(0, CHUNK_ROWS)], buf, sem).wait()

    lane = lax.iota(jnp.int32, 16)

    def compute(g, buf):
        base_idx = (g * CHUNK_B + lane) * L

        def idf_body(l, s):
            return s + plsc.load_gather(idf_v, [base_idx + l])

        s_vec = lax.fori_loop(0, L, idf_body, jnp.zeros((16,), jnp.float32))
        s_buf[0:16] = s_vec

        def batch_body(bi, _):
            s = plsc.load_gather(s_buf, [jnp.broadcast_to(bi, (16,))])

            acc0 = jnp.zeros((16,), jnp.float32)
            acc1 = jnp.zeros((16,), jnp.float32)
            base = bi * L
            for l in range(L):
                acc0 = acc0 + buf[base + l, 0:16]
                acc1 = acc1 + buf[base + l, 16:32]

            ob = g * CHUNK_B + bi
            out_v[ob, 0:16] = acc0 * s
            out_v[ob, 16:32] = acc1 * s
            return ()

        lax.fori_loop(0, CHUNK_B, batch_body, ())

    issue(0, rows_a, sem_a)
    issue(1, rows_b, sem_b)

    def ring_body(g2, _):
        for bsel in range(2):
            g = g2 * 2 + bsel
            buf = rows_a if bsel == 0 else rows_b
            sem = sem_a if bsel == 0 else sem_b
            drain(buf, sem)
            compute(g, buf)

            @pl.when(g + 2 < NCHUNKS)
            def _():
                issue(g + 2, buf, sem)
        return ()

    lax.fori_loop(0, NCHUNKS // 2, ring_body, ())

    pltpu.sync_copy(out_v, out_hbm.at[pl.ds(wid * BPW, BPW)])


def kernel(inputs, idf, weights):
    idx = inputs.astype(jnp.int32).reshape(B * L // ROWS_PER_GATHER,
                                           ROWS_PER_GATHER)
    return _sc_embed(idx, idf.reshape(B * L), weights)


# speedup vs baseline: 2.9760x; 1.0241x over previous
import functools

import jax
import jax.numpy as jnp
from jax import lax
from jax.experimental import pallas as pl
from jax.experimental.pallas import tpu as pltpu
from jax.experimental.pallas import tpu_sc as plsc

B, L, V, D = 16384, 50, 1000000, 32

NC, NS = 2, 16
NW = NC * NS
BPW = B // NW
CB = 128
NCHUNKS = BPW // CB

_mesh = plsc.VectorSubcoreMesh(core_axis_name="c", subcore_axis_name="s")


@functools.partial(
    pl.kernel,
    out_type=jax.ShapeDtypeStruct((B, D), jnp.float32),
    mesh=_mesh,
    compiler_params=pltpu.CompilerParams(needs_layout_passes=False,
                                         use_tc_tiling_on_sc=False),
    scratch_types=[
        pltpu.VMEM((L, BPW), jnp.int32),
        pltpu.VMEM((BPW * L,), jnp.float32),
        pltpu.VMEM((CB, D), jnp.float32),
        pltpu.VMEM((CB, D), jnp.float32),
        pltpu.VMEM((BPW, D), jnp.float32),
        pltpu.VMEM((16,), jnp.float32),
        pltpu.SemaphoreType.DMA,
        pltpu.SemaphoreType.DMA,
    ],
)
def _sc_embed(idxT_hbm, idf_hbm, w_hbm, out_hbm,
              idxT_v, idf_v, acc_a, acc_b, out_v, s_buf, sem_a, sem_b):
    wid = lax.axis_index("s") * NC + lax.axis_index("c")

    pltpu.sync_copy(idxT_hbm.at[:, pl.ds(wid * BPW, BPW)], idxT_v)
    pltpu.sync_copy(idf_hbm.at[pl.ds(wid * (BPW * L), BPW * L)], idf_v)

    bufs = [(acc_a, sem_a), (acc_b, sem_b)]

    def issue_l0(c, buf, sem):
        return pltpu.async_copy(
            w_hbm.at[idxT_v.at[0, pl.ds(c * CB, CB)]], buf, sem)

    def issue_adds(c, buf, sem):
        return [
            pltpu.async_copy(
                w_hbm.at[idxT_v.at[l, pl.ds(c * CB, CB)]], buf, sem,
                add=True)
            for l in range(1, L)
        ]

    lane = lax.iota(jnp.int32, 16)

    def compute(c, buf):
        def group_body(gr, _):
            base_idx = (c * CB + gr * 16 + lane) * L

            def idf_body(l, s):
                return s + plsc.load_gather(idf_v, [base_idx + l])

            s_vec = lax.fori_loop(0, L, idf_body,
                                  jnp.zeros((16,), jnp.float32))
            s_buf[0:16] = s_vec

            def scale_body(bi, _):
                s = plsc.load_gather(s_buf, [jnp.broadcast_to(bi, (16,))])
                b = gr * 16 + bi
                ob = c * CB + b
                out_v[ob, 0:16] = buf[b, 0:16] * s
                out_v[ob, 16:32] = buf[b, 16:32] * s
                return ()

            lax.fori_loop(0, 16, scale_body, ())
            return ()

        lax.fori_loop(0, CB // 16, group_body, ())

    d_l0 = {0: issue_l0(0, *bufs[0]), 1: issue_l0(1, *bufs[1])}
    d_l0[0].wait()
    d_add = {0: issue_adds(0, *bufs[0])}

    for c in range(NCHUNKS):
        buf, sem = bufs[c % 2]
        if c + 1 < NCHUNKS:
            d_l0[c + 1].wait()
            d_add[c + 1] = issue_adds(c + 1, *bufs[(c + 1) % 2])
        for d in d_add[c]:
            d.wait()
        compute(c, buf)
        if c + 2 < NCHUNKS:
            d_l0[c + 2] = issue_l0(c + 2, buf, sem)

    pltpu.sync_copy(out_v, out_hbm.at[pl.ds(wid * BPW, BPW)])


def kernel(inputs, idf, weights):
    idx_t = jnp.transpose(inputs.astype(jnp.int32))
    return _sc_embed(idx_t, idf.reshape(B * L), weights)
